# Initial kernel scaffold; baseline (speedup 1.0000x reference)
import jax
import jax.numpy as jnp
from jax.experimental import pallas as pl


def kernel(drug_feat, gene_feat, ei_dd, ei_gg, ei_dt, ei_td, W0dd, b0dd, W0gg, b0gg, W0dt, b0dt, W0td, b0td, W1dd, b1dd, W1gg, b1gg, W1dt, b1dt, W1td, b1td):
    raise NotImplementedError("write your pallas kernel here")


# speedup vs baseline: 18.0639x; 18.0639x over previous
import functools

import jax
import jax.numpy as jnp
from jax import lax
from jax.experimental import pallas as pl
from jax.experimental.pallas import tpu as pltpu
from jax.experimental.pallas import tpu_sc as plsc

N = 10000
E = 320000
D_OUT = 64
NW = 32
EPW = E // NW
CH = 125
NCH = EPW // CH
NP = 10240
RPW = NP // 16
HTOT = 6 * N
HSL = HTOT // 15

_mesh = plsc.VectorSubcoreMesh(core_axis_name="c", subcore_axis_name="s")


@functools.partial(
    pl.kernel,
    out_type=jax.ShapeDtypeStruct((2 * HTOT,), jnp.float32),
    mesh=_mesh,
    scratch_types=[
        pltpu.VMEM((NCH, CH), jnp.int32),
        pltpu.VMEM((128,), jnp.float32),
        pltpu.VMEM((HSL,), jnp.float32),
        pltpu.VMEM_SHARED((HTOT,), jnp.float32),
    ],
)
def _sc_hist(idx_hbm, zeros_hbm, ones_hbm, out_hbm, idx_v, ones_v, hv,
             hist_sh):
    c = lax.axis_index("c")
    s = lax.axis_index("s")
    wid = s * 2 + c
    pltpu.sync_copy(ones_hbm, ones_v)

    @pl.when(s < 15)
    def _zero():
        pltpu.sync_copy(zeros_hbm, hv)
        pltpu.sync_copy(hv, hist_sh.at[pl.ds(s * HSL, HSL)])

    plsc.subcore_barrier()
    for a in range(6):
        pltpu.sync_copy(idx_hbm.at[a, wid], idx_v)

        def _body(cc, carry):
            pltpu.sync_copy(ones_v.at[pl.ds(0, CH)],
                            hist_sh.at[idx_v.at[cc]], add=True)
            return carry

        lax.fori_loop(0, NCH, _body, 0)
    plsc.subcore_barrier()

    @pl.when(s < 15)
    def _flush():
        pltpu.sync_copy(hist_sh.at[pl.ds(s * HSL, HSL)], hv)
        pltpu.sync_copy(hv, out_hbm.at[pl.ds(c * HTOT + s * HSL, HSL)])


@functools.partial(
    pl.kernel,
    out_type=jax.ShapeDtypeStruct((2, 4, NP, D_OUT), jnp.float32),
    mesh=_mesh,
    compiler_params=pltpu.CompilerParams(use_tc_tiling_on_sc=False),
    scratch_types=[
        pltpu.VMEM((NCH, CH), jnp.int32),
        pltpu.VMEM((NCH, CH), jnp.int32),
        pltpu.VMEM((CH, D_OUT), jnp.float32),
        pltpu.VMEM_SHARED((NP, D_OUT), jnp.float32),
        pltpu.SemaphoreType.DMA,
    ],
)
def _sc_edge(hp_hbm, src_hbm, dst_hbm, zrows_hbm, acc_hbm,
             src_v, dst_v, rows_v, acc_sh, sem):
    c = lax.axis_index("c")
    s = lax.axis_index("s")
    wid = s * 2 + c
    for r in range(4):
        plsc.subcore_barrier()
        pltpu.sync_copy(zrows_hbm, acc_sh.at[pl.ds(s * RPW, RPW)])
        plsc.subcore_barrier()
        pltpu.sync_copy(src_hbm.at[r, wid], src_v)
        pltpu.sync_copy(dst_hbm.at[r, wid], dst_v)

        def _body(cc, carry):
            pltpu.async_copy(hp_hbm.at[src_v.at[cc]], rows_v, sem).wait()
            pltpu.sync_copy(rows_v, acc_sh.at[dst_v.at[cc]], add=True)
            return carry

        lax.fori_loop(0, NCH, _body, 0)
        plsc.subcore_barrier()
        pltpu.sync_copy(acc_sh.at[pl.ds(s * RPW, RPW)],
                        acc_hbm.at[c, r, pl.ds(s * RPW, RPW)])


def _scale_body(hist_ref, pre_ref, post_ref):
    h = hist_ref[0] + hist_ref[1]
    dd = lax.rsqrt(h[0:2] + 1.0)
    gs = jnp.where(h[2:4] > 0, lax.rsqrt(h[2:4]), 0.0)
    gd = jnp.where(h[4:6] > 0, lax.rsqrt(h[4:6]), 0.0)
    pre_ref[...] = jnp.concatenate([dd, gs], axis=0)
    post_ref[...] = jnp.concatenate([dd, gd], axis=0)


def _scale_call(hist2):
    return pl.pallas_call(
        _scale_body,
        in_specs=[pl.BlockSpec((2, 6, N), lambda: (0, 0, 0))],
        out_specs=[pl.BlockSpec((4, N), lambda: (0, 0)),
                   pl.BlockSpec((4, N), lambda: (0, 0))],
        out_shape=[jax.ShapeDtypeStruct((4, N), jnp.float32),
                   jax.ShapeDtypeStruct((4, N), jnp.float32)],
    )(hist2)


def _pre_body(xs_ref, m_ref, w_ref, s_ref, o_ref):
    x = xs_ref[0]
    m = m_ref[0]
    w = w_ref[0]
    sv = s_ref[0]
    h = jnp.dot(x * m, w, preferred_element_type=jnp.float32)
    o_ref[0] = h * sv


def _pre_call(xs, masks, wst, pres, din):
    R = 1000
    return pl.pallas_call(
        _pre_body,
        grid=(4, N // R),
        in_specs=[
            pl.BlockSpec((1, R, din), lambda r, i: (r % 2, i, 0)),
            pl.BlockSpec((1, R, din), lambda r, i: (r, i, 0)),
            pl.BlockSpec((1, din, D_OUT), lambda r, i: (r, 0, 0)),
            pl.BlockSpec((1, R, 1), lambda r, i: (r, i, 0)),
        ],
        out_specs=pl.BlockSpec((1, R, D_OUT), lambda r, i: (r, i, 0)),
        out_shape=jax.ShapeDtypeStruct((4, N, D_OUT), jnp.float32),
    )(xs, masks, wst, pres)


def _post_body(acc_ref, hp_ref, ps_ref, b_ref, xd_ref, xg_ref):
    acc = acc_ref[0] + acc_ref[1]
    ps = ps_ref[...]
    b = b_ref[...]

    def norm(t):
        t = jnp.maximum(t, 0.0)
        n = jnp.sqrt(jnp.sum(t * t, axis=-1, keepdims=True))
        return t / jnp.maximum(n, 1e-12)

    od = norm(ps[0] * (acc[0] + hp_ref[0]) + b[0])
    og = norm(ps[1] * (acc[1] + hp_ref[1]) + b[1])
    odt = norm(ps[2] * acc[2] + b[2])
    otd = norm(ps[3] * acc[3] + b[3])
    xd_ref[...] = od + otd
    xg_ref[...] = og + odt


def _post_call(acc, hp, posts, bst):
    R = 1000
    return pl.pallas_call(
        _post_body,
        grid=(N // R,),
        in_specs=[
            pl.BlockSpec((2, 4, R, D_OUT), lambda i: (0, 0, i, 0)),
            pl.BlockSpec((2, R, D_OUT), lambda i: (0, i, 0)),
            pl.BlockSpec((4, R, 1), lambda i: (0, i, 0)),
            pl.BlockSpec((4, 1, D_OUT), lambda i: (0, 0, 0)),
        ],
        out_specs=[pl.BlockSpec((R, D_OUT), lambda i: (i, 0)),
                   pl.BlockSpec((R, D_OUT), lambda i: (i, 0))],
        out_shape=[jax.ShapeDtypeStruct((N, D_OUT), jnp.float32),
                   jax.ShapeDtypeStruct((N, D_OUT), jnp.float32)],
    )(acc, hp, posts, bst)


def _mask(c, shape):
    keep = jax.random.bernoulli(jax.random.fold_in(jax.random.key(42), c),
                                0.8, shape)
    return jnp.where(keep, jnp.float32(1.0 / 0.8), jnp.float32(0.0))


def kernel(drug_feat, gene_feat, ei_dd, ei_gg, ei_dt, ei_td,
           W0dd, b0dd, W0gg, b0gg, W0dt, b0dt, W0td, b0td,
           W1dd, b1dd, W1gg, b1gg, W1dt, b1dt, W1td, b1td):
    idx6 = jnp.stack([ei_dd[1], ei_gg[1], ei_dt[0], ei_td[0],
                      ei_dt[1], ei_td[1]])
    idx6_off = (idx6 + jnp.arange(6, dtype=jnp.int32)[:, None] * N)
    idx6_off = idx6_off.reshape(6, NW, NCH, CH)
    esrc = jnp.stack([ei_dd[0], ei_gg[0] + N, ei_dt[0] + 2 * N,
                      ei_td[0] + 3 * N]).reshape(4, NW, NCH, CH)
    edst = jnp.stack([ei_dd[1], ei_gg[1], ei_dt[1],
                      ei_td[1]]).reshape(4, NW, NCH, CH)
    z_hist = jnp.zeros((HSL,), jnp.float32)
    ones_h = jnp.ones((128,), jnp.float32)
    z_rows = jnp.zeros((RPW, D_OUT), jnp.float32)

    hist = _sc_hist(idx6_off, z_hist, ones_h)
    pre, post = _scale_call(hist.reshape(2, 6, N))
    pres = pre.reshape(4, N, 1)
    posts = post.reshape(4, N, 1)

    masks = [jnp.stack([_mask(0, (N, 128)), _mask(1, (N, 128)),
                        _mask(2, (N, 128)), _mask(3, (N, 128))]),
             jnp.stack([_mask(4, (N, D_OUT)), _mask(5, (N, D_OUT)),
                        _mask(6, (N, D_OUT)), _mask(7, (N, D_OUT))])]
    wst = [jnp.stack([W0dd, W0gg, W0dt, W0td]),
           jnp.stack([W1dd, W1gg, W1dt, W1td])]
    bst = [jnp.stack([b0dd, b0gg, b0dt, b0td]).reshape(4, 1, D_OUT),
           jnp.stack([b1dd, b1gg, b1dt, b1td]).reshape(4, 1, D_OUT)]

    xs = jnp.stack([drug_feat, gene_feat])
    for l in range(2):
        din = 128 if l == 0 else D_OUT
        hp = _pre_call(xs, masks[l], wst[l], pres, din)
        acc = _sc_edge(hp.reshape(4 * N, D_OUT), esrc, edst, z_rows)
        xd, xg = _post_call(acc[:, :, :N, :], hp, posts, bst[l])
        xs = jnp.stack([xd, xg])
    return xd, xg


# speedup vs baseline: 22.1619x; 1.2269x over previous
import functools

import jax
import jax.numpy as jnp
from jax import lax
from jax.experimental import pallas as pl
from jax.experimental.pallas import tpu as pltpu
from jax.experimental.pallas import tpu_sc as plsc

N = 10000
E = 320000
D_OUT = 64
NW = 32
EPW = E // NW
CH = 125
NCH = EPW // CH
NP = 10240
RPW = NP // 16
HTOT = 6 * N
HSL = HTOT // 15

_mesh = plsc.VectorSubcoreMesh(core_axis_name="c", subcore_axis_name="s")


@functools.partial(
    pl.kernel,
    out_type=jax.ShapeDtypeStruct((2 * HTOT,), jnp.float32),
    mesh=_mesh,
    scratch_types=[
        pltpu.VMEM((NCH, CH), jnp.int32),
        pltpu.VMEM((128,), jnp.float32),
        pltpu.VMEM((HSL,), jnp.float32),
        pltpu.VMEM_SHARED((HTOT,), jnp.float32),
    ],
)
def _sc_hist(idx_hbm, zeros_hbm, ones_hbm, out_hbm, idx_v, ones_v, hv,
             hist_sh):
    c = lax.axis_index("c")
    s = lax.axis_index("s")
    wid = s * 2 + c
    pltpu.sync_copy(ones_hbm, ones_v)

    @pl.when(s < 15)
    def _zero():
        pltpu.sync_copy(zeros_hbm, hv)
        pltpu.sync_copy(hv, hist_sh.at[pl.ds(s * HSL, HSL)])

    plsc.subcore_barrier()
    for a in range(6):
        pltpu.sync_copy(idx_hbm.at[a, wid], idx_v)

        def _body(cc, carry):
            pltpu.sync_copy(ones_v.at[pl.ds(0, CH)],
                            hist_sh.at[idx_v.at[cc]], add=True)
            return carry

        lax.fori_loop(0, NCH, _body, 0)
    plsc.subcore_barrier()

    @pl.when(s < 15)
    def _flush():
        pltpu.sync_copy(hist_sh.at[pl.ds(s * HSL, HSL)], hv)
        pltpu.sync_copy(hv, out_hbm.at[pl.ds(c * HTOT + s * HSL, HSL)])


@functools.partial(
    pl.kernel,
    out_type=jax.ShapeDtypeStruct((2, 4, NP, D_OUT), jnp.float32),
    mesh=_mesh,
    compiler_params=pltpu.CompilerParams(use_tc_tiling_on_sc=False),
    scratch_types=[
        pltpu.VMEM((NCH, CH), jnp.int32),
        pltpu.VMEM((NCH, CH), jnp.int32),
        pltpu.VMEM((4, CH, D_OUT), jnp.float32),
        pltpu.VMEM_SHARED((NP, D_OUT), jnp.float32),
        pltpu.SemaphoreType.DMA,
    ],
)
def _sc_edge(hp_hbm, src_hbm, dst_hbm, zrows_hbm, acc_hbm,
             src_v, dst_v, rows_v, acc_sh, sem):
    c = lax.axis_index("c")
    s = lax.axis_index("s")
    wid = s * 2 + c
    for r in range(4):
        plsc.subcore_barrier()
        pltpu.sync_copy(zrows_hbm, acc_sh.at[pl.ds(s * RPW, RPW)])
        plsc.subcore_barrier()
        pltpu.sync_copy(src_hbm.at[r, wid], src_v)
        pltpu.sync_copy(dst_hbm.at[r, wid], dst_v)

        def _body(g, carry):
            base = g * 4
            hs = [pltpu.async_copy(hp_hbm.at[src_v.at[base + j]],
                                   rows_v.at[j], sem) for j in range(4)]
            for j in range(4):
                hs[j].wait()
            for j in range(4):
                pltpu.sync_copy(rows_v.at[j], acc_sh.at[dst_v.at[base + j]],
                                add=True)
            return carry

        lax.fori_loop(0, NCH // 4, _body, 0)
        plsc.subcore_barrier()
        pltpu.sync_copy(acc_sh.at[pl.ds(s * RPW, RPW)],
                        acc_hbm.at[c, r, pl.ds(s * RPW, RPW)])


def _scale_body(hist_ref, pre_ref, post_ref):
    h = hist_ref[0] + hist_ref[1]
    dd = lax.rsqrt(h[0:2] + 1.0)
    gs = jnp.where(h[2:4] > 0, lax.rsqrt(h[2:4]), 0.0)
    gd = jnp.where(h[4:6] > 0, lax.rsqrt(h[4:6]), 0.0)
    pre_ref[...] = jnp.concatenate([dd, gs], axis=0)
    post_ref[...] = jnp.concatenate([dd, gd], axis=0)


def _scale_call(hist2):
    return pl.pallas_call(
        _scale_body,
        in_specs=[pl.BlockSpec((2, 6, N), lambda: (0, 0, 0))],
        out_specs=[pl.BlockSpec((4, N), lambda: (0, 0)),
                   pl.BlockSpec((4, N), lambda: (0, 0))],
        out_shape=[jax.ShapeDtypeStruct((4, N), jnp.float32),
                   jax.ShapeDtypeStruct((4, N), jnp.float32)],
    )(hist2)


def _pre_body(xs_ref, m_ref, w_ref, s_ref, o_ref):
    x = xs_ref[0]
    m = m_ref[0]
    w = w_ref[0]
    sv = s_ref[0]
    h = jnp.dot(x * m, w, preferred_element_type=jnp.float32)
    o_ref[0] = h * sv


def _pre_call(xs, masks, wst, pres, din):
    R = 1000
    return pl.pallas_call(
        _pre_body,
        grid=(4, N // R),
        in_specs=[
            pl.BlockSpec((1, R, din), lambda r, i: (r % 2, i, 0)),
            pl.BlockSpec((1, R, din), lambda r, i: (r, i, 0)),
            pl.BlockSpec((1, din, D_OUT), lambda r, i: (r, 0, 0)),
            pl.BlockSpec((1, R, 1), lambda r, i: (r, i, 0)),
        ],
        out_specs=pl.BlockSpec((1, R, D_OUT), lambda r, i: (r, i, 0)),
        out_shape=jax.ShapeDtypeStruct((4, N, D_OUT), jnp.float32),
    )(xs, masks, wst, pres)


def _post_body(acc_ref, hp_ref, ps_ref, b_ref, xd_ref, xg_ref):
    acc = acc_ref[0] + acc_ref[1]
    ps = ps_ref[...]
    b = b_ref[...]

    def norm(t):
        t = jnp.maximum(t, 0.0)
        n = jnp.sqrt(jnp.sum(t * t, axis=-1, keepdims=True))
        return t / jnp.maximum(n, 1e-12)

    od = norm(ps[0] * (acc[0] + hp_ref[0]) + b[0])
    og = norm(ps[1] * (acc[1] + hp_ref[1]) + b[1])
    odt = norm(ps[2] * acc[2] + b[2])
    otd = norm(ps[3] * acc[3] + b[3])
    xd_ref[...] = od + otd
    xg_ref[...] = og + odt


def _post_call(acc, hp, posts, bst):
    R = 1000
    return pl.pallas_call(
        _post_body,
        grid=(N // R,),
        in_specs=[
            pl.BlockSpec((2, 4, R, D_OUT), lambda i: (0, 0, i, 0)),
            pl.BlockSpec((2, R, D_OUT), lambda i: (0, i, 0)),
            pl.BlockSpec((4, R, 1), lambda i: (0, i, 0)),
            pl.BlockSpec((4, 1, D_OUT), lambda i: (0, 0, 0)),
        ],
        out_specs=[pl.BlockSpec((R, D_OUT), lambda i: (i, 0)),
                   pl.BlockSpec((R, D_OUT), lambda i: (i, 0))],
        out_shape=[jax.ShapeDtypeStruct((N, D_OUT), jnp.float32),
                   jax.ShapeDtypeStruct((N, D_OUT), jnp.float32)],
    )(acc, hp, posts, bst)


def _mask(c, shape):
    keep = jax.random.bernoulli(jax.random.fold_in(jax.random.key(42), c),
                                0.8, shape)
    return jnp.where(keep, jnp.float32(1.0 / 0.8), jnp.float32(0.0))


def kernel(drug_feat, gene_feat, ei_dd, ei_gg, ei_dt, ei_td,
           W0dd, b0dd, W0gg, b0gg, W0dt, b0dt, W0td, b0td,
           W1dd, b1dd, W1gg, b1gg, W1dt, b1dt, W1td, b1td):
    idx6 = jnp.stack([ei_dd[1], ei_gg[1], ei_dt[0], ei_td[0],
                      ei_dt[1], ei_td[1]])
    idx6_off = (idx6 + jnp.arange(6, dtype=jnp.int32)[:, None] * N)
    idx6_off = idx6_off.reshape(6, NW, NCH, CH)
    esrc = jnp.stack([ei_dd[0], ei_gg[0] + N, ei_dt[0] + 2 * N,
                      ei_td[0] + 3 * N]).reshape(4, NW, NCH, CH)
    edst = jnp.stack([ei_dd[1], ei_gg[1], ei_dt[1],
                      ei_td[1]]).reshape(4, NW, NCH, CH)
    z_hist = jnp.zeros((HSL,), jnp.float32)
    ones_h = jnp.ones((128,), jnp.float32)
    z_rows = jnp.zeros((RPW, D_OUT), jnp.float32)

    hist = _sc_hist(idx6_off, z_hist, ones_h)
    pre, post = _scale_call(hist.reshape(2, 6, N))
    pres = pre.reshape(4, N, 1)
    posts = post.reshape(4, N, 1)

    masks = [jnp.stack([_mask(0, (N, 128)), _mask(1, (N, 128)),
                        _mask(2, (N, 128)), _mask(3, (N, 128))]),
             jnp.stack([_mask(4, (N, D_OUT)), _mask(5, (N, D_OUT)),
                        _mask(6, (N, D_OUT)), _mask(7, (N, D_OUT))])]
    wst = [jnp.stack([W0dd, W0gg, W0dt, W0td]),
           jnp.stack([W1dd, W1gg, W1dt, W1td])]
    bst = [jnp.stack([b0dd, b0gg, b0dt, b0td]).reshape(4, 1, D_OUT),
           jnp.stack([b1dd, b1gg, b1dt, b1td]).reshape(4, 1, D_OUT)]

    xs = jnp.stack([drug_feat, gene_feat])
    for l in range(2):
        din = 128 if l == 0 else D_OUT
        hp = _pre_call(xs, masks[l], wst[l], pres, din)
        acc = _sc_edge(hp.reshape(4 * N, D_OUT), esrc, edst, z_rows)
        xd, xg = _post_call(acc[:, :, :N, :], hp, posts, bst[l])
        xs = jnp.stack([xd, xg])
    return xd, xg


# speedup vs baseline: 25.4776x; 1.1496x over previous
import functools

import jax
import jax.numpy as jnp
from jax import lax
from jax.experimental import pallas as pl
from jax.experimental.pallas import tpu as pltpu
from jax.experimental.pallas import tpu_sc as plsc

N = 10000
E = 320000
D_OUT = 64
NW = 32
EPW = E // NW
CH = 125
NCH = EPW // CH
NP = 10240
RPW = NP // 16
HTOT = 6 * N
HSL = HTOT // 15

_mesh = plsc.VectorSubcoreMesh(core_axis_name="c", subcore_axis_name="s")


@functools.partial(
    pl.kernel,
    out_type=jax.ShapeDtypeStruct((2 * HTOT,), jnp.float32),
    mesh=_mesh,
    scratch_types=[
        pltpu.VMEM((NCH, CH), jnp.int32),
        pltpu.VMEM((128,), jnp.float32),
        pltpu.VMEM((HSL,), jnp.float32),
        pltpu.VMEM_SHARED((HTOT,), jnp.float32),
    ],
)
def _sc_hist(idx_hbm, zeros_hbm, ones_hbm, out_hbm, idx_v, ones_v, hv,
             hist_sh):
    c = lax.axis_index("c")
    s = lax.axis_index("s")
    wid = s * 2 + c
    pltpu.sync_copy(ones_hbm, ones_v)

    @pl.when(s < 15)
    def _zero():
        pltpu.sync_copy(zeros_hbm, hv)
        pltpu.sync_copy(hv, hist_sh.at[pl.ds(s * HSL, HSL)])

    plsc.subcore_barrier()
    for a in range(6):
        pltpu.sync_copy(idx_hbm.at[a, wid], idx_v)

        def _body(cc, carry):
            pltpu.sync_copy(ones_v.at[pl.ds(0, CH)],
                            hist_sh.at[idx_v.at[cc]], add=True)
            return carry

        lax.fori_loop(0, NCH, _body, 0)
    plsc.subcore_barrier()

    @pl.when(s < 15)
    def _flush():
        pltpu.sync_copy(hist_sh.at[pl.ds(s * HSL, HSL)], hv)
        pltpu.sync_copy(hv, out_hbm.at[pl.ds(c * HTOT + s * HSL, HSL)])


@functools.partial(
    pl.kernel,
    out_type=jax.ShapeDtypeStruct((2, 4, NP, D_OUT), jnp.float32),
    mesh=_mesh,
    compiler_params=pltpu.CompilerParams(use_tc_tiling_on_sc=False),
    scratch_types=[
        pltpu.VMEM((NCH, CH), jnp.int32),
        pltpu.VMEM((NCH, CH), jnp.int32),
        pltpu.VMEM((8, CH, D_OUT), jnp.float32),
        pltpu.VMEM_SHARED((NP, D_OUT), jnp.float32),
        pltpu.SemaphoreType.DMA,
    ],
)
def _sc_edge(hp_hbm, src_hbm, dst_hbm, zrows_hbm, acc_hbm,
             src_v, dst_v, rows_v, acc_sh, sem):
    c = lax.axis_index("c")
    s = lax.axis_index("s")
    wid = s * 2 + c
    for r in range(4):
        plsc.subcore_barrier()
        pltpu.sync_copy(zrows_hbm, acc_sh.at[pl.ds(s * RPW, RPW)])
        plsc.subcore_barrier()
        pltpu.sync_copy(src_hbm.at[r, wid], src_v)
        pltpu.sync_copy(dst_hbm.at[r, wid], dst_v)

        ng = NCH // 4
        for j in range(4):
            pltpu.async_copy(hp_hbm.at[src_v.at[j]], rows_v.at[j], sem)

        def _body(g, carry):
            cur = (g % 2) * 4
            nxt = 4 - cur
            for j in range(4):
                pltpu.make_async_copy(zrows_hbm.at[pl.ds(0, CH)],
                                      rows_v.at[cur + j], sem).wait()

            @pl.when(g < ng - 1)
            def _fire_next():
                for j in range(4):
                    pltpu.async_copy(hp_hbm.at[src_v.at[(g + 1) * 4 + j]],
                                     rows_v.at[nxt + j], sem)

            for j in range(4):
                pltpu.sync_copy(rows_v.at[cur + j],
                                acc_sh.at[dst_v.at[g * 4 + j]], add=True)
            return carry

        lax.fori_loop(0, ng, _body, 0)
        plsc.subcore_barrier()
        pltpu.sync_copy(acc_sh.at[pl.ds(s * RPW, RPW)],
                        acc_hbm.at[c, r, pl.ds(s * RPW, RPW)])


def _scale_body(hist_ref, pre_ref, post_ref):
    h = hist_ref[0] + hist_ref[1]
    dd = lax.rsqrt(h[0:2] + 1.0)
    gs = jnp.where(h[2:4] > 0, lax.rsqrt(h[2:4]), 0.0)
    gd = jnp.where(h[4:6] > 0, lax.rsqrt(h[4:6]), 0.0)
    pre_ref[...] = jnp.concatenate([dd, gs], axis=0)
    post_ref[...] = jnp.concatenate([dd, gd], axis=0)


def _scale_call(hist2):
    return pl.pallas_call(
        _scale_body,
        in_specs=[pl.BlockSpec((2, 6, N), lambda: (0, 0, 0))],
        out_specs=[pl.BlockSpec((4, N), lambda: (0, 0)),
                   pl.BlockSpec((4, N), lambda: (0, 0))],
        out_shape=[jax.ShapeDtypeStruct((4, N), jnp.float32),
                   jax.ShapeDtypeStruct((4, N), jnp.float32)],
    )(hist2)


def _pre_body(xs_ref, m_ref, w_ref, s_ref, o_ref):
    x = xs_ref[0]
    m = m_ref[0]
    w = w_ref[0]
    sv = s_ref[0]
    h = jnp.dot(x * m, w, preferred_element_type=jnp.float32)
    o_ref[0] = h * sv


def _pre_call(xs, masks, wst, pres, din):
    R = 1000
    return pl.pallas_call(
        _pre_body,
        grid=(4, N // R),
        in_specs=[
            pl.BlockSpec((1, R, din), lambda r, i: (r % 2, i, 0)),
            pl.BlockSpec((1, R, din), lambda r, i: (r, i, 0)),
            pl.BlockSpec((1, din, D_OUT), lambda r, i: (r, 0, 0)),
            pl.BlockSpec((1, R, 1), lambda r, i: (r, i, 0)),
        ],
        out_specs=pl.BlockSpec((1, R, D_OUT), lambda r, i: (r, i, 0)),
        out_shape=jax.ShapeDtypeStruct((4, N, D_OUT), jnp.float32),
    )(xs, masks, wst, pres)


def _post_body(acc_ref, hp_ref, ps_ref, b_ref, xd_ref, xg_ref):
    acc = acc_ref[0] + acc_ref[1]
    ps = ps_ref[...]
    b = b_ref[...]

    def norm(t):
        t = jnp.maximum(t, 0.0)
        n = jnp.sqrt(jnp.sum(t * t, axis=-1, keepdims=True))
        return t / jnp.maximum(n, 1e-12)

    od = norm(ps[0] * (acc[0] + hp_ref[0]) + b[0])
    og = norm(ps[1] * (acc[1] + hp_ref[1]) + b[1])
    odt = norm(ps[2] * acc[2] + b[2])
    otd = norm(ps[3] * acc[3] + b[3])
    xd_ref[...] = od + otd
    xg_ref[...] = og + odt


def _post_call(acc, hp, posts, bst):
    R = 1000
    return pl.pallas_call(
        _post_body,
        grid=(N // R,),
        in_specs=[
            pl.BlockSpec((2, 4, R, D_OUT), lambda i: (0, 0, i, 0)),
            pl.BlockSpec((2, R, D_OUT), lambda i: (0, i, 0)),
            pl.BlockSpec((4, R, 1), lambda i: (0, i, 0)),
            pl.BlockSpec((4, 1, D_OUT), lambda i: (0, 0, 0)),
        ],
        out_specs=[pl.BlockSpec((R, D_OUT), lambda i: (i, 0)),
                   pl.BlockSpec((R, D_OUT), lambda i: (i, 0))],
        out_shape=[jax.ShapeDtypeStruct((N, D_OUT), jnp.float32),
                   jax.ShapeDtypeStruct((N, D_OUT), jnp.float32)],
    )(acc, hp, posts, bst)


def _mask(c, shape):
    keep = jax.random.bernoulli(jax.random.fold_in(jax.random.key(42), c),
                                0.8, shape)
    return jnp.where(keep, jnp.float32(1.0 / 0.8), jnp.float32(0.0))


def kernel(drug_feat, gene_feat, ei_dd, ei_gg, ei_dt, ei_td,
           W0dd, b0dd, W0gg, b0gg, W0dt, b0dt, W0td, b0td,
           W1dd, b1dd, W1gg, b1gg, W1dt, b1dt, W1td, b1td):
    idx6 = jnp.stack([ei_dd[1], ei_gg[1], ei_dt[0], ei_td[0],
                      ei_dt[1], ei_td[1]])
    idx6_off = (idx6 + jnp.arange(6, dtype=jnp.int32)[:, None] * N)
    idx6_off = idx6_off.reshape(6, NW, NCH, CH)
    esrc = jnp.stack([ei_dd[0], ei_gg[0] + N, ei_dt[0] + 2 * N,
                      ei_td[0] + 3 * N]).reshape(4, NW, NCH, CH)
    edst = jnp.stack([ei_dd[1], ei_gg[1], ei_dt[1],
                      ei_td[1]]).reshape(4, NW, NCH, CH)
    z_hist = jnp.zeros((HSL,), jnp.float32)
    ones_h = jnp.ones((128,), jnp.float32)
    z_rows = jnp.zeros((RPW, D_OUT), jnp.float32)

    hist = _sc_hist(idx6_off, z_hist, ones_h)
    pre, post = _scale_call(hist.reshape(2, 6, N))
    pres = pre.reshape(4, N, 1)
    posts = post.reshape(4, N, 1)

    masks = [jnp.stack([_mask(0, (N, 128)), _mask(1, (N, 128)),
                        _mask(2, (N, 128)), _mask(3, (N, 128))]),
             jnp.stack([_mask(4, (N, D_OUT)), _mask(5, (N, D_OUT)),
                        _mask(6, (N, D_OUT)), _mask(7, (N, D_OUT))])]
    wst = [jnp.stack([W0dd, W0gg, W0dt, W0td]),
           jnp.stack([W1dd, W1gg, W1dt, W1td])]
    bst = [jnp.stack([b0dd, b0gg, b0dt, b0td]).reshape(4, 1, D_OUT),
           jnp.stack([b1dd, b1gg, b1dt, b1td]).reshape(4, 1, D_OUT)]

    xs = jnp.stack([drug_feat, gene_feat])
    for l in range(2):
        din = 128 if l == 0 else D_OUT
        hp = _pre_call(xs, masks[l], wst[l], pres, din)
        acc = _sc_edge(hp.reshape(4 * N, D_OUT), esrc, edst, z_rows)
        xd, xg = _post_call(acc[:, :, :N, :], hp, posts, bst[l])
        xs = jnp.stack([xd, xg])
    return xd, xg


# speedup vs baseline: 29.6818x; 1.1650x over previous
import functools

import jax
import jax.numpy as jnp
import numpy as np
from jax import lax
from jax.experimental import pallas as pl
from jax.experimental.pallas import tpu as pltpu
from jax.experimental.pallas import tpu_sc as plsc

N = 10000
E = 320000
D_OUT = 64
NW = 32
EPW = E // NW
CH = 125
NCH = EPW // CH
NP = 10240
RPW = NP // 16
HTOT = 6 * N
HSL = HTOT // 15

_mesh = plsc.VectorSubcoreMesh(core_axis_name="c", subcore_axis_name="s")


@functools.partial(
    pl.kernel,
    out_type=jax.ShapeDtypeStruct((2 * HTOT,), jnp.float32),
    mesh=_mesh,
    scratch_types=[
        pltpu.VMEM((NCH, CH), jnp.int32),
        pltpu.VMEM((128,), jnp.float32),
        pltpu.VMEM((HSL,), jnp.float32),
        pltpu.VMEM_SHARED((HTOT,), jnp.float32),
    ],
)
def _sc_hist(idx_hbm, zeros_hbm, ones_hbm, out_hbm, idx_v, ones_v, hv,
             hist_sh):
    c = lax.axis_index("c")
    s = lax.axis_index("s")
    wid = s * 2 + c
    pltpu.sync_copy(ones_hbm, ones_v)

    @pl.when(s < 15)
    def _zero():
        pltpu.sync_copy(zeros_hbm, hv)
        pltpu.sync_copy(hv, hist_sh.at[pl.ds(s * HSL, HSL)])

    plsc.subcore_barrier()
    for a in range(6):
        pltpu.sync_copy(idx_hbm.at[a, wid], idx_v)

        def _body(cc, carry):
            pltpu.sync_copy(ones_v.at[pl.ds(0, CH)],
                            hist_sh.at[idx_v.at[cc]], add=True)
            return carry

        lax.fori_loop(0, NCH, _body, 0)
    plsc.subcore_barrier()

    @pl.when(s < 15)
    def _flush():
        pltpu.sync_copy(hist_sh.at[pl.ds(s * HSL, HSL)], hv)
        pltpu.sync_copy(hv, out_hbm.at[pl.ds(c * HTOT + s * HSL, HSL)])


@functools.partial(
    pl.kernel,
    out_type=jax.ShapeDtypeStruct((2, 4, NP, D_OUT), jnp.float32),
    mesh=_mesh,
    compiler_params=pltpu.CompilerParams(use_tc_tiling_on_sc=False),
    scratch_types=[
        pltpu.VMEM((NCH, CH), jnp.int32),
        pltpu.VMEM((NCH, CH), jnp.int32),
        pltpu.VMEM((8, CH, D_OUT), jnp.float32),
        pltpu.VMEM_SHARED((NP, D_OUT), jnp.float32),
        pltpu.SemaphoreType.DMA,
    ],
)
def _sc_edge(hp_hbm, src_hbm, dst_hbm, zrows_hbm, acc_hbm,
             src_v, dst_v, rows_v, acc_sh, sem):
    c = lax.axis_index("c")
    s = lax.axis_index("s")
    wid = s * 2 + c
    for r in range(4):
        plsc.subcore_barrier()
        pltpu.sync_copy(zrows_hbm, acc_sh.at[pl.ds(s * RPW, RPW)])
        plsc.subcore_barrier()
        pltpu.sync_copy(src_hbm.at[r, wid], src_v)
        pltpu.sync_copy(dst_hbm.at[r, wid], dst_v)

        ng = NCH // 4
        for j in range(4):
            pltpu.async_copy(hp_hbm.at[src_v.at[j]], rows_v.at[j], sem)

        def _body(g, carry):
            cur = (g % 2) * 4
            nxt = 4 - cur
            for j in range(4):
                pltpu.make_async_copy(zrows_hbm.at[pl.ds(0, CH)],
                                      rows_v.at[cur + j], sem).wait()

            @pl.when(g < ng - 1)
            def _fire_next():
                for j in range(4):
                    pltpu.async_copy(hp_hbm.at[src_v.at[(g + 1) * 4 + j]],
                                     rows_v.at[nxt + j], sem)

            for j in range(4):
                pltpu.sync_copy(rows_v.at[cur + j],
                                acc_sh.at[dst_v.at[g * 4 + j]], add=True)
            return carry

        lax.fori_loop(0, ng, _body, 0)
        plsc.subcore_barrier()
        pltpu.sync_copy(acc_sh.at[pl.ds(s * RPW, RPW)],
                        acc_hbm.at[c, r, pl.ds(s * RPW, RPW)])


def _scale_body(hist_ref, pre_ref, post_ref):
    h = hist_ref[0] + hist_ref[1]
    dd = lax.rsqrt(h[0:2] + 1.0)
    gs = jnp.where(h[2:4] > 0, lax.rsqrt(h[2:4]), 0.0)
    gd = jnp.where(h[4:6] > 0, lax.rsqrt(h[4:6]), 0.0)
    pre_ref[...] = jnp.concatenate([dd, gs], axis=0)
    post_ref[...] = jnp.concatenate([dd, gd], axis=0)


def _scale_call(hist2):
    return pl.pallas_call(
        _scale_body,
        in_specs=[pl.BlockSpec((2, 6, N), lambda: (0, 0, 0))],
        out_specs=[pl.BlockSpec((4, N), lambda: (0, 0)),
                   pl.BlockSpec((4, N), lambda: (0, 0))],
        out_shape=[jax.ShapeDtypeStruct((4, N), jnp.float32),
                   jax.ShapeDtypeStruct((4, N), jnp.float32)],
    )(hist2)


def _pre_body(xs_ref, m_ref, w_ref, s_ref, o_ref):
    x = xs_ref[0]
    m = m_ref[0]
    w = w_ref[0]
    sv = s_ref[0]
    h = jnp.dot(x * m, w, preferred_element_type=jnp.float32)
    o_ref[0] = h * sv


def _pre_call(xs, masks, wst, pres, din):
    R = 1000
    return pl.pallas_call(
        _pre_body,
        grid=(4, N // R),
        in_specs=[
            pl.BlockSpec((1, R, din), lambda r, i: (r % 2, i, 0)),
            pl.BlockSpec((1, R, din), lambda r, i: (r, i, 0)),
            pl.BlockSpec((1, din, D_OUT), lambda r, i: (r, 0, 0)),
            pl.BlockSpec((1, R, 1), lambda r, i: (r, i, 0)),
        ],
        out_specs=pl.BlockSpec((1, R, D_OUT), lambda r, i: (r, i, 0)),
        out_shape=jax.ShapeDtypeStruct((4, N, D_OUT), jnp.float32),
    )(xs, masks, wst, pres)


def _post_body(acc_ref, hp_ref, ps_ref, b_ref, xd_ref, xg_ref):
    acc = acc_ref[0] + acc_ref[1]
    ps = ps_ref[...]
    b = b_ref[...]

    def norm(t):
        t = jnp.maximum(t, 0.0)
        n = jnp.sqrt(jnp.sum(t * t, axis=-1, keepdims=True))
        return t / jnp.maximum(n, 1e-12)

    od = norm(ps[0] * (acc[0] + hp_ref[0]) + b[0])
    og = norm(ps[1] * (acc[1] + hp_ref[1]) + b[1])
    odt = norm(ps[2] * acc[2] + b[2])
    otd = norm(ps[3] * acc[3] + b[3])
    xd_ref[...] = od + otd
    xg_ref[...] = og + odt


def _post_call(acc, hp, posts, bst):
    R = 1000
    return pl.pallas_call(
        _post_body,
        grid=(N // R,),
        in_specs=[
            pl.BlockSpec((2, 4, R, D_OUT), lambda i: (0, 0, i, 0)),
            pl.BlockSpec((2, R, D_OUT), lambda i: (0, i, 0)),
            pl.BlockSpec((4, R, 1), lambda i: (0, i, 0)),
            pl.BlockSpec((4, 1, D_OUT), lambda i: (0, 0, 0)),
        ],
        out_specs=[pl.BlockSpec((R, D_OUT), lambda i: (i, 0)),
                   pl.BlockSpec((R, D_OUT), lambda i: (i, 0))],
        out_shape=[jax.ShapeDtypeStruct((N, D_OUT), jnp.float32),
                   jax.ShapeDtypeStruct((N, D_OUT), jnp.float32)],
    )(acc, hp, posts, bst)


def _mask(c, shape):
    keep = jax.random.bernoulli(jax.random.fold_in(jax.random.key(42), c),
                                0.8, shape)
    return np.asarray(jnp.where(keep, jnp.float32(1.0 / 0.8),
                                jnp.float32(0.0)))


_MASKS = [np.stack([_mask(c, (N, 128)) for c in range(4)]),
          np.stack([_mask(c, (N, D_OUT)) for c in range(4, 8)])]


def kernel(drug_feat, gene_feat, ei_dd, ei_gg, ei_dt, ei_td,
           W0dd, b0dd, W0gg, b0gg, W0dt, b0dt, W0td, b0td,
           W1dd, b1dd, W1gg, b1gg, W1dt, b1dt, W1td, b1td):
    idx6 = jnp.stack([ei_dd[1], ei_gg[1], ei_dt[0], ei_td[0],
                      ei_dt[1], ei_td[1]])
    idx6_off = (idx6 + jnp.arange(6, dtype=jnp.int32)[:, None] * N)
    idx6_off = idx6_off.reshape(6, NW, NCH, CH)
    esrc = jnp.stack([ei_dd[0], ei_gg[0] + N, ei_dt[0] + 2 * N,
                      ei_td[0] + 3 * N]).reshape(4, NW, NCH, CH)
    edst = jnp.stack([ei_dd[1], ei_gg[1], ei_dt[1],
                      ei_td[1]]).reshape(4, NW, NCH, CH)
    z_hist = jnp.zeros((HSL,), jnp.float32)
    ones_h = jnp.ones((128,), jnp.float32)
    z_rows = jnp.zeros((RPW, D_OUT), jnp.float32)

    hist = _sc_hist(idx6_off, z_hist, ones_h)
    pre, post = _scale_call(hist.reshape(2, 6, N))
    pres = pre.reshape(4, N, 1)
    posts = post.reshape(4, N, 1)

    masks = [jnp.asarray(_MASKS[0]), jnp.asarray(_MASKS[1])]
    wst = [jnp.stack([W0dd, W0gg, W0dt, W0td]),
           jnp.stack([W1dd, W1gg, W1dt, W1td])]
    bst = [jnp.stack([b0dd, b0gg, b0dt, b0td]).reshape(4, 1, D_OUT),
           jnp.stack([b1dd, b1gg, b1dt, b1td]).reshape(4, 1, D_OUT)]

    xs = jnp.stack([drug_feat, gene_feat])
    for l in range(2):
        din = 128 if l == 0 else D_OUT
        hp = _pre_call(xs, masks[l], wst[l], pres, din)
        acc = _sc_edge(hp.reshape(4 * N, D_OUT), esrc, edst, z_rows)
        xd, xg = _post_call(acc, hp, posts, bst[l])
        xs = jnp.stack([xd, xg])
    return xd, xg
